# Initial kernel scaffold; baseline (speedup 1.0000x reference)
import jax
import jax.numpy as jnp
from jax.experimental import pallas as pl


def kernel(x, Wg, bg, W1, b1, W2, b2):
    raise NotImplementedError("write your pallas kernel here")


# speedup vs baseline: 1.0483x; 1.0483x over previous
import jax
import jax.numpy as jnp
from jax.experimental import pallas as pl
from jax.experimental.pallas import tpu as pltpu

N_TOK = 4096
D = 1024
H = 2048
C = 1024
E = 8
BN = 1024


def _moe_block_kernel(x_ref, wg_ref, bg_ref, w1_ref, b1_ref, w2_ref, b2_ref,
                      out_ref, xh_ref, gate_ref):
    j = pl.program_id(1)

    @pl.when(j == 0)
    def _gate():
        x = x_ref[...]
        logits = jax.lax.dot_general(
            x, wg_ref[...], (((1,), (0,)), ((), ())),
            preferred_element_type=jnp.float32) + bg_ref[...]
        lane = jax.lax.broadcasted_iota(jnp.int32, logits.shape, 1)
        m1 = jnp.max(logits, axis=1, keepdims=True)
        i1 = jnp.min(jnp.where(logits == m1, lane, E), axis=1, keepdims=True)
        masked = jnp.where(lane == i1, -jnp.inf, logits)
        m2 = jnp.max(masked, axis=1, keepdims=True)
        i2 = jnp.min(jnp.where(masked == m2, lane, E), axis=1, keepdims=True)
        w1g = 1.0 / (1.0 + jnp.exp(m2 - m1))
        w2g = 1.0 - w1g
        gate_ref[...] = (jnp.where(lane == i1, w1g, 0.0)
                         + jnp.where(lane == i2, w2g, 0.0))
        xh_ref[...] = x.astype(jnp.bfloat16)

    xh = xh_ref[...]
    h = jnp.dot(xh, w1_ref[0], preferred_element_type=jnp.float32)
    h = jnp.maximum(h + b1_ref[0], 0.0)
    o = jnp.dot(h.astype(jnp.bfloat16), w2_ref[0],
                preferred_element_type=jnp.float32)
    o = o + b2_ref[0]
    lane_e = jax.lax.broadcasted_iota(jnp.int32, (BN, E), 1)
    g = jnp.sum(jnp.where(lane_e == j, gate_ref[...], 0.0), axis=1,
                keepdims=True)
    contrib = g * o

    @pl.when(j == 0)
    def _init():
        out_ref[...] = contrib

    @pl.when(j > 0)
    def _acc():
        out_ref[...] += contrib


def kernel(x, Wg, bg, W1, b1, W2, b2):
    out = pl.pallas_call(
        _moe_block_kernel,
        grid=(N_TOK // BN, E),
        in_specs=[
            pl.BlockSpec((BN, D), lambda i, j: (i, 0)),
            pl.BlockSpec((D, E), lambda i, j: (0, 0)),
            pl.BlockSpec((1, E), lambda i, j: (0, 0)),
            pl.BlockSpec((1, D, H), lambda i, j: (j, 0, 0)),
            pl.BlockSpec((1, 1, H), lambda i, j: (j, 0, 0)),
            pl.BlockSpec((1, H, C), lambda i, j: (j, 0, 0)),
            pl.BlockSpec((1, 1, C), lambda i, j: (j, 0, 0)),
        ],
        out_specs=pl.BlockSpec((BN, C), lambda i, j: (i, 0)),
        out_shape=jax.ShapeDtypeStruct((N_TOK, C), jnp.float32),
        scratch_shapes=[
            pltpu.VMEM((BN, D), jnp.bfloat16),
            pltpu.VMEM((BN, E), jnp.float32),
        ],
    )(x, Wg, bg.reshape(1, E), W1.astype(jnp.bfloat16), b1.reshape(E, 1, H),
      W2.astype(jnp.bfloat16), b2.reshape(E, 1, C))
    return out


# speedup vs baseline: 1.3321x; 1.2707x over previous
import functools

import jax
import jax.numpy as jnp
from jax import lax
from jax.experimental import pallas as pl
from jax.experimental.pallas import tpu as pltpu
from jax.experimental.pallas import tpu_sc as plsc

N_TOK = 4096
D = 1024
H = 2048
C = 1024
E = 8
BT = 256
PTOT = 2 * N_TOK + E * BT
NBLK = PTOT // BT
CB = 512
NC, NS = 2, 16
NW = NC * NS
TPW = N_TOK // NW
HT = TPW // 2
SUBT = 32


def _router_kernel(x_ref, wg_ref, bg_ref,
                   p0_ref, p1_ref, w0_ref, w1_ref, be_ref):
    x = x_ref[...]
    logits = jax.lax.dot_general(
        x, wg_ref[...], (((1,), (0,)), ((), ())),
        preferred_element_type=jnp.float32) + bg_ref[...]
    lane = lax.broadcasted_iota(jnp.int32, (N_TOK, E), 1)
    m1 = jnp.max(logits, axis=1, keepdims=True)
    i1 = jnp.min(jnp.where(logits == m1, lane, E), axis=1, keepdims=True)
    masked = jnp.where(lane == i1, -jnp.inf, logits)
    m2 = jnp.max(masked, axis=1, keepdims=True)
    i2 = jnp.min(jnp.where(masked == m2, lane, E), axis=1, keepdims=True)
    e1 = jnp.exp(m2 - m1)
    den = 1.0 + e1
    w0_ref[...] = 1.0 / den
    w1_ref[...] = e1 / den

    ind = jnp.logical_or(lane == i1, lane == i2).astype(jnp.bfloat16)
    rio = lax.broadcasted_iota(jnp.int32, (CB, CB), 0)
    cio = lax.broadcasted_iota(jnp.int32, (CB, CB), 1)
    tri = (rio > cio).astype(jnp.bfloat16)
    carry = jnp.zeros((1, E), jnp.float32)
    excs = []
    for b in range(N_TOK // CB):
        blk = ind[b * CB:(b + 1) * CB, :]
        excs.append(jnp.dot(tri, blk, preferred_element_type=jnp.float32)
                    + carry)
        carry = carry + jnp.sum(blk.astype(jnp.float32), axis=0,
                                keepdims=True)
    exc = jnp.concatenate(excs, axis=0)
    tot = carry
    padded = jnp.ceil(tot * (1.0 / BT)) * BT
    r8 = lax.broadcasted_iota(jnp.int32, (E, E), 0)
    c8 = lax.broadcasted_iota(jnp.int32, (E, E), 1)
    upper = (r8 < c8).astype(jnp.bfloat16)
    off = jnp.dot(padded.astype(jnp.bfloat16), upper,
                  preferred_element_type=jnp.float32)
    ends = off + padded

    pos0 = jnp.zeros((N_TOK, 1), jnp.float32)
    pos1 = jnp.zeros((N_TOK, 1), jnp.float32)
    for e in range(E):
        slot_e = off[:, e:e + 1] + exc[:, e:e + 1]
        pos0 = pos0 + jnp.where(i1 == e, slot_e, 0.0)
        pos1 = pos1 + jnp.where(i2 == e, slot_e, 0.0)
    p0_ref[...] = pos0.astype(jnp.int32)
    p1_ref[...] = pos1.astype(jnp.int32)

    g_iota = (lax.broadcasted_iota(jnp.int32, (1, NBLK), 1)
              .astype(jnp.float32) * float(BT))
    bexp = jnp.zeros((1, NBLK), jnp.float32)
    for e in range(E):
        bexp = bexp + (g_iota >= ends[:, e:e + 1]).astype(jnp.float32)
    be_ref[...] = jnp.minimum(bexp, float(E - 1)).astype(jnp.int32)


def _ffn_kernel(be_ref, xs_ref, w1_ref, b1_ref, w2_ref, b2_ref, ws_ref,
                ys_ref):
    h = jnp.dot(xs_ref[...], w1_ref[0], preferred_element_type=jnp.float32)
    h = jnp.maximum(h + b1_ref[0], 0.0)
    o = jnp.dot(h, w2_ref[0], preferred_element_type=jnp.float32)
    ys_ref[...] = (o + b2_ref[0]) * ws_ref[...]


def _sc_scatter_impl(x_hbm, p0_hbm, p1_hbm, w0_hbm, w1_hbm, xs_hbm, ws_hbm,
                     xbuf, pbuf, wbuf, sem0, sem1):
    wid = lax.axis_index("s") * NC + lax.axis_index("c")
    n0 = wid * TPW
    pltpu.sync_copy(p0_hbm.at[pl.ds(n0, HT)], pbuf.at[0])
    pltpu.sync_copy(p0_hbm.at[pl.ds(n0 + HT, HT)], pbuf.at[1])
    pltpu.sync_copy(p1_hbm.at[pl.ds(n0, HT)], pbuf.at[2])
    pltpu.sync_copy(p1_hbm.at[pl.ds(n0 + HT, HT)], pbuf.at[3])
    pltpu.sync_copy(w0_hbm.at[pl.ds(n0, HT)], wbuf.at[0])
    pltpu.sync_copy(w0_hbm.at[pl.ds(n0 + HT, HT)], wbuf.at[1])
    pltpu.sync_copy(w1_hbm.at[pl.ds(n0, HT)], wbuf.at[2])
    pltpu.sync_copy(w1_hbm.at[pl.ds(n0 + HT, HT)], wbuf.at[3])
    for t in range(2):
        pltpu.sync_copy(x_hbm.at[pl.ds(n0 + HT * t, HT)], xbuf)
        c0 = pltpu.async_copy(xbuf, xs_hbm.at[pbuf.at[t]], sem0)
        c1 = pltpu.async_copy(xbuf, xs_hbm.at[pbuf.at[2 + t]], sem1)
        c0.wait()
        c1.wait()
    for q in range(4):
        cw = pltpu.async_copy(wbuf.at[q], ws_hbm.at[pbuf.at[q]], sem0)
        cw.wait()


def _sc_combine_impl(ys_hbm, p0_hbm, p1_hbm, out_hbm, i0, i1b, r0, r1,
                     semA, semB):
    wid = lax.axis_index("s") * NC + lax.axis_index("c")
    n0 = wid * TPW

    def body(t, _):
        base = n0 + t * SUBT
        pltpu.sync_copy(p0_hbm.at[pl.ds(base, SUBT)], i0)
        pltpu.sync_copy(p1_hbm.at[pl.ds(base, SUBT)], i1b)
        cA = pltpu.async_copy(ys_hbm.at[i0], r0, semA)
        cB = pltpu.async_copy(ys_hbm.at[i1b], r1, semB)
        cA.wait()
        cB.wait()

        def row(i, _):
            for cc in range(C // 16):
                sl = pl.ds(cc * 16, 16)
                r0[i, sl] = r0[i, sl] + r1[i, sl]
            return 0

        lax.fori_loop(0, SUBT, row, 0)
        pltpu.sync_copy(r0, out_hbm.at[pl.ds(base, SUBT)])
        return 0

    lax.fori_loop(0, TPW // SUBT, body, 0)


@functools.cache
def _get_sc_kernels():
    mesh = plsc.VectorSubcoreMesh(
        core_axis_name="c", subcore_axis_name="s",
        num_cores=NC, num_subcores=NS)
    sc_scatter = pl.kernel(
        _sc_scatter_impl,
        out_type=[
            jax.ShapeDtypeStruct((PTOT, D), jnp.float32),
            jax.ShapeDtypeStruct((PTOT,), jnp.float32),
        ],
        mesh=mesh,
        scratch_types=[
            pltpu.VMEM((HT, D), jnp.float32),
            pltpu.VMEM((4, HT), jnp.int32),
            pltpu.VMEM((4, HT), jnp.float32),
            pltpu.SemaphoreType.DMA,
            pltpu.SemaphoreType.DMA,
        ],
    )
    sc_combine = pl.kernel(
        _sc_combine_impl,
        out_type=jax.ShapeDtypeStruct((N_TOK, C), jnp.float32),
        mesh=mesh,
        scratch_types=[
            pltpu.VMEM((SUBT,), jnp.int32),
            pltpu.VMEM((SUBT,), jnp.int32),
            pltpu.VMEM((SUBT, C), jnp.float32),
            pltpu.VMEM((SUBT, C), jnp.float32),
            pltpu.SemaphoreType.DMA,
            pltpu.SemaphoreType.DMA,
        ],
    )
    return sc_scatter, sc_combine


def _router(x, Wg, bg):
    return pl.pallas_call(
        _router_kernel,
        out_shape=[
            jax.ShapeDtypeStruct((N_TOK, 1), jnp.int32),
            jax.ShapeDtypeStruct((N_TOK, 1), jnp.int32),
            jax.ShapeDtypeStruct((N_TOK, 1), jnp.float32),
            jax.ShapeDtypeStruct((N_TOK, 1), jnp.float32),
            jax.ShapeDtypeStruct((1, NBLK), jnp.int32),
        ],
    )(x, Wg, bg.reshape(1, E))


def _ffn(be, xs, W1, b1, W2, b2, ws):
    grid_spec = pltpu.PrefetchScalarGridSpec(
        num_scalar_prefetch=1,
        grid=(NBLK,),
        in_specs=[
            pl.BlockSpec((BT, D), lambda g, be: (g, 0)),
            pl.BlockSpec((1, D, H), lambda g, be: (be[g], 0, 0)),
            pl.BlockSpec((1, 1, H), lambda g, be: (be[g], 0, 0)),
            pl.BlockSpec((1, H, C), lambda g, be: (be[g], 0, 0)),
            pl.BlockSpec((1, 1, C), lambda g, be: (be[g], 0, 0)),
            pl.BlockSpec((BT, 1), lambda g, be: (g, 0)),
        ],
        out_specs=pl.BlockSpec((BT, C), lambda g, be: (g, 0)),
    )
    return pl.pallas_call(
        _ffn_kernel,
        grid_spec=grid_spec,
        out_shape=jax.ShapeDtypeStruct((PTOT, C), jnp.float32),
    )(be, xs, W1, b1.reshape(E, 1, H), W2, b2.reshape(E, 1, C), ws)


def kernel(x, Wg, bg, W1, b1, W2, b2):
    sc_scatter, sc_combine = _get_sc_kernels()
    p0, p1, w0, w1, be = _router(x, Wg, bg)
    p0f = p0.reshape(N_TOK)
    p1f = p1.reshape(N_TOK)
    xs, ws = sc_scatter(x, p0f, p1f, w0.reshape(N_TOK), w1.reshape(N_TOK))
    ys = _ffn(be.reshape(NBLK), xs, W1, b1, W2, b2, ws.reshape(PTOT, 1))
    return sc_combine(ys, p0f, p1f)
